# baseline (speedup 1.0000x reference)
import functools

import jax
import jax.numpy as jnp
from jax import lax
from jax.experimental import pallas as pl
from jax.experimental.pallas import tpu as pltpu
from jax.experimental.pallas import tpu_sc as plsc

D = 128
LANES = 16
CHUNK = 64
NBUF = 4
NBLK = D // LANES

_GDN = lax.GatherDimensionNumbers(
    offset_dims=(), collapsed_slice_dims=(0,), start_index_map=(0,))


def _lane_shuffle(x, idx):
    return lax.gather(x, idx[:, None], _GDN, (1,),
                      mode=lax.GatherScatterMode.PROMISE_IN_BOUNDS)


def _row_partial(erows, trows, row):
    prods = []
    for b in range(NBLK):
        e = erows[row, pl.ds(b * LANES, LANES)]
        t = trows[row, pl.ds(b * LANES, LANES)]
        prods.append(e * t)
    while len(prods) > 1:
        prods = [prods[i] + prods[i + 1] for i in range(0, len(prods), 2)]
    return prods[0]


def _tree_reduce(vs):
    iota = jnp.arange(LANES, dtype=jnp.int32)
    s = 1
    while len(vs) > 1:
        mask = (iota & s) == 0
        perm = iota ^ s
        nxt = []
        for i in range(0, len(vs), 2):
            a, b = vs[i], vs[i + 1]
            keep = jnp.where(mask, a, b)
            other = _lane_shuffle(jnp.where(mask, b, a), perm)
            nxt.append(keep + other)
        vs = nxt
        s *= 2
    return vs[0]


def _sc_body(num_cores):
    def body(ent_idx_hbm, type_idx_hbm, ent_hbm, type_hbm, out_hbm,
             idx_e, idx_t, erows, trows, outv, *sems):
        wid = lax.axis_index("s") * num_cores + lax.axis_index("c")
        nch = idx_e.shape[0]
        sem_e = sems[:NBUF]
        sem_t = sems[NBUF:]
        pltpu.sync_copy(ent_idx_hbm.at[wid], idx_e)
        pltpu.sync_copy(type_idx_hbm.at[wid], idx_t)

        def start(j):
            p = j % NBUF
            he = pltpu.async_copy(ent_hbm.at[idx_e.at[j]], erows.at[p], sem_e[p])
            ht = pltpu.async_copy(type_hbm.at[idx_t.at[j]], trows.at[p], sem_t[p])
            return he, ht

        handles = {}
        for j in range(min(NBUF, nch)):
            handles[j] = start(j)
        for j in range(nch):
            p = j % NBUF
            he, ht = handles.pop(j)
            he.wait()
            ht.wait()
            ebuf = erows.at[p]
            tbuf = trows.at[p]

            def group_body(g, _, ebuf=ebuf, tbuf=tbuf, j=j):
                base = g * LANES
                vs = [_row_partial(ebuf, tbuf, base + r) for r in range(LANES)]
                outv[pl.ds(j * CHUNK + base, LANES)] = _tree_reduce(vs)
                return 0

            lax.fori_loop(0, CHUNK // LANES, group_body, 0)
            if j + NBUF < nch:
                handles[j + NBUF] = start(j + NBUF)
        pltpu.sync_copy(outv, out_hbm.at[wid])

    return body


def kernel(entity, pos_type, ent_emb, type_embedding):
    B = entity.shape[0]
    info = plsc.get_sparse_core_info()
    nw = info.num_cores * info.num_subcores
    bpw = B // nw
    nch = bpw // CHUNK
    mesh = plsc.VectorSubcoreMesh(core_axis_name="c", subcore_axis_name="s")
    ent_idx = entity.astype(jnp.int32).reshape(nw, nch, CHUNK)
    type_idx = pos_type.astype(jnp.int32).reshape(nw, nch, CHUNK)
    k = functools.partial(
        pl.kernel,
        mesh=mesh,
        compiler_params=pltpu.CompilerParams(needs_layout_passes=False),
        out_type=jax.ShapeDtypeStruct((nw, bpw), jnp.float32),
        scratch_types=[
            pltpu.VMEM((nch, CHUNK), jnp.int32),
            pltpu.VMEM((nch, CHUNK), jnp.int32),
            pltpu.VMEM((NBUF, CHUNK, D), jnp.float32),
            pltpu.VMEM((NBUF, CHUNK, D), jnp.float32),
            pltpu.VMEM((bpw,), jnp.float32),
        ] + [pltpu.SemaphoreType.DMA] * (2 * NBUF),
    )(_sc_body(info.num_cores))
    out = k(ent_idx, type_idx, ent_emb, type_embedding)
    return out.reshape(B, 1)


# speedup vs baseline: 1.1270x; 1.1270x over previous
import functools

import jax
import jax.numpy as jnp
from jax import lax
from jax.experimental import pallas as pl
from jax.experimental.pallas import tpu as pltpu
from jax.experimental.pallas import tpu_sc as plsc

D = 128
LANES = 16
CHUNK = 128
NBUF = 3
NBLK = D // LANES

_GDN = lax.GatherDimensionNumbers(
    offset_dims=(), collapsed_slice_dims=(0,), start_index_map=(0,))


def _lane_shuffle(x, idx):
    return lax.gather(x, idx[:, None], _GDN, (1,),
                      mode=lax.GatherScatterMode.PROMISE_IN_BOUNDS)


def _row_partial(erows, trows, row):
    prods = []
    for b in range(NBLK):
        e = erows[row, pl.ds(b * LANES, LANES)]
        t = trows[row, pl.ds(b * LANES, LANES)]
        prods.append(e * t)
    while len(prods) > 1:
        prods = [prods[i] + prods[i + 1] for i in range(0, len(prods), 2)]
    return prods[0]


def _tree_reduce(vs):
    iota = jnp.arange(LANES, dtype=jnp.int32)
    s = 1
    while len(vs) > 1:
        mask = (iota & s) == 0
        perm = iota ^ s
        nxt = []
        for i in range(0, len(vs), 2):
            a, b = vs[i], vs[i + 1]
            keep = jnp.where(mask, a, b)
            other = _lane_shuffle(jnp.where(mask, b, a), perm)
            nxt.append(keep + other)
        vs = nxt
        s *= 2
    return vs[0]


def _sc_body(num_cores):
    def body(ent_idx_hbm, type_idx_hbm, ent_hbm, type_hbm, out_hbm,
             idx_e, idx_t, erows, trows, outv, type_sp, *sems):
        wid = lax.axis_index("s") * num_cores + lax.axis_index("c")
        sid = lax.axis_index("s")
        nch = idx_e.shape[0]
        sem_e = sems[:NBUF]
        sem_t = sems[NBUF:2 * NBUF]
        sem_tbl = sems[2 * NBUF]
        pltpu.sync_copy(ent_idx_hbm.at[wid], idx_e)
        pltpu.sync_copy(type_idx_hbm.at[wid], idx_t)

        def start_e(j):
            p = j % NBUF
            return pltpu.async_copy(ent_hbm.at[idx_e.at[j]], erows.at[p], sem_e[p])

        def start_t(j):
            p = j % NBUF
            return pltpu.async_copy(type_sp.at[idx_t.at[j]], trows.at[p], sem_t[p])

        eh = {}
        th = {}
        for j in range(min(NBUF, nch)):
            eh[j] = start_e(j)

        @pl.when(sid == 0)
        def _():
            pltpu.async_copy(type_hbm, type_sp, sem_tbl).wait()

        plsc.subcore_barrier()
        for j in range(min(NBUF, nch)):
            th[j] = start_t(j)

        for j in range(nch):
            p = j % NBUF
            eh.pop(j).wait()
            th.pop(j).wait()
            ebuf = erows.at[p]
            tbuf = trows.at[p]

            def group_body(g, _, ebuf=ebuf, tbuf=tbuf, j=j):
                base = g * LANES
                vs = [_row_partial(ebuf, tbuf, base + r) for r in range(LANES)]
                outv[pl.ds(j * CHUNK + base, LANES)] = _tree_reduce(vs)
                return 0

            lax.fori_loop(0, CHUNK // LANES, group_body, 0)
            if j + NBUF < nch:
                eh[j + NBUF] = start_e(j + NBUF)
                th[j + NBUF] = start_t(j + NBUF)
        pltpu.sync_copy(outv, out_hbm.at[wid])

    return body


def kernel(entity, pos_type, ent_emb, type_embedding):
    B = entity.shape[0]
    info = plsc.get_sparse_core_info()
    nw = info.num_cores * info.num_subcores
    bpw = B // nw
    nch = bpw // CHUNK
    mesh = plsc.VectorSubcoreMesh(core_axis_name="c", subcore_axis_name="s")
    ent_idx = entity.astype(jnp.int32).reshape(nw, nch, CHUNK)
    type_idx = pos_type.astype(jnp.int32).reshape(nw, nch, CHUNK)
    k = functools.partial(
        pl.kernel,
        mesh=mesh,
        compiler_params=pltpu.CompilerParams(needs_layout_passes=False),
        out_type=jax.ShapeDtypeStruct((nw, bpw), jnp.float32),
        scratch_types=[
            pltpu.VMEM((nch, CHUNK), jnp.int32),
            pltpu.VMEM((nch, CHUNK), jnp.int32),
            pltpu.VMEM((NBUF, CHUNK, D), jnp.float32),
            pltpu.VMEM((NBUF, CHUNK, D), jnp.float32),
            pltpu.VMEM((bpw,), jnp.float32),
            pltpu.VMEM_SHARED(type_embedding.shape, jnp.float32),
        ] + [pltpu.SemaphoreType.DMA] * (2 * NBUF + 1),
    )(_sc_body(info.num_cores))
    out = k(ent_idx, type_idx, ent_emb, type_embedding)
    return out.reshape(B, 1)


# speedup vs baseline: 1.1731x; 1.0409x over previous
import functools

import jax
import jax.numpy as jnp
import numpy as np
from jax import lax
from jax.experimental import pallas as pl
from jax.experimental.pallas import tpu as pltpu
from jax.experimental.pallas import tpu_sc as plsc

D = 128
LANES = 16
CHUNK = 128
NBUF = 3
NBLK = D // LANES
WPR = D // 2

_GDN = lax.GatherDimensionNumbers(
    offset_dims=(), collapsed_slice_dims=(0,), start_index_map=(0,))


def _lane_shuffle(x, idx):
    return lax.gather(x, idx[:, None], _GDN, (1,),
                      mode=lax.GatherScatterMode.PROMISE_IN_BOUNDS)


def _tree_reduce(vs):
    iota = jnp.arange(LANES, dtype=jnp.int32)
    s = 1
    while len(vs) > 1:
        mask = (iota & s) == 0
        perm = iota ^ s
        nxt = []
        for i in range(0, len(vs), 2):
            a, b = vs[i], vs[i + 1]
            keep = jnp.where(mask, a, b)
            other = _lane_shuffle(jnp.where(mask, b, a), perm)
            nxt.append(keep + other)
        vs = nxt
        s *= 2
    return vs[0]


def _sc_body(num_cores):
    def body(idx_hbm, ent_hbm, tblw_hbm, out_hbm,
             idx, erows, tblw, outv, sem_tbl, *sems):
        iota = jnp.arange(LANES, dtype=jnp.int32)
        offs = [iota + LANES * b2 for b2 in range(WPR // LANES)]
        splats = [jnp.full((LANES,), r, jnp.int32) for r in range(LANES)]
        wid = lax.axis_index("s") * num_cores + lax.axis_index("c")
        nch = idx.shape[1]
        pltpu.sync_copy(idx_hbm.at[wid], idx)
        tbl_h = pltpu.async_copy(tblw_hbm, tblw, sem_tbl)

        def start_e(j):
            return pltpu.async_copy(
                ent_hbm.at[idx.at[0, j]], erows.at[j % NBUF], sems[j % NBUF])

        eh = {}
        for j in range(min(NBUF, nch)):
            eh[j] = start_e(j)
        tbl_h.wait()
        tflat = tblw

        for j in range(nch):
            eh.pop(j).wait()
            ebuf = erows.at[j % NBUF]

            def group_body(g, _, ebuf=ebuf, j=j, offs=offs, splats=splats):
                base = g * LANES
                ti = idx[1, j, pl.ds(base, LANES)]
                ws = ti * WPR
                vs = []
                for r in range(LANES):
                    wsr = _lane_shuffle(ws, splats[r])
                    tb = []
                    for b2 in range(WPR // LANES):
                        w = plsc.load_gather(tflat, [wsr + offs[b2]])
                        pair = plsc.unpack(plsc.bitcast(w, jnp.bfloat16),
                                           format=plsc.PackFormat.INTERLEAVED)
                        tb.extend(pair)
                    prods = []
                    for b in range(NBLK):
                        e = ebuf[base + r, pl.ds(b * LANES, LANES)]
                        prods.append(e * tb[b])
                    while len(prods) > 1:
                        prods = [prods[i] + prods[i + 1]
                                 for i in range(0, len(prods), 2)]
                    vs.append(prods[0])
                outv[pl.ds(base, LANES)] = _tree_reduce(vs)
                return 0

            lax.fori_loop(0, CHUNK // LANES, group_body, 0)
            if j + NBUF < nch:
                eh[j + NBUF] = start_e(j + NBUF)
            pltpu.sync_copy(outv, out_hbm.at[wid, pl.ds(j * CHUNK, CHUNK)])

    return body


def _pack_type_table(type_embedding):
    nt = type_embedding.shape[0]
    perm = np.empty((D,), np.int32)
    for p in range(D // 32):
        for i in range(16):
            perm[32 * p + 2 * i] = 32 * p + i
            perm[32 * p + 2 * i + 1] = 32 * p + 16 + i
    shuf = type_embedding.astype(jnp.bfloat16)[:, perm]
    return lax.bitcast_convert_type(
        shuf.reshape(nt, WPR, 2), jnp.int32).reshape(nt * WPR)


def kernel(entity, pos_type, ent_emb, type_embedding):
    B = entity.shape[0]
    info = plsc.get_sparse_core_info()
    nw = info.num_cores * info.num_subcores
    bpw = B // nw
    nch = bpw // CHUNK
    mesh = plsc.VectorSubcoreMesh(core_axis_name="c", subcore_axis_name="s")
    idx = jnp.stack(
        [entity.astype(jnp.int32).reshape(nw, nch, CHUNK),
         pos_type.astype(jnp.int32).reshape(nw, nch, CHUNK)], axis=1)
    tblw = _pack_type_table(type_embedding)
    k = functools.partial(
        pl.kernel,
        mesh=mesh,
        compiler_params=pltpu.CompilerParams(needs_layout_passes=False),
        out_type=jax.ShapeDtypeStruct((nw, bpw), jnp.float32),
        scratch_types=[
            pltpu.VMEM((2, nch, CHUNK), jnp.int32),
            pltpu.VMEM((NBUF, CHUNK, D), jnp.float32),
            pltpu.VMEM((tblw.shape[0],), jnp.int32),
            pltpu.VMEM((CHUNK,), jnp.float32),
            pltpu.SemaphoreType.DMA,
        ] + [pltpu.SemaphoreType.DMA] * NBUF,
    )(_sc_body(info.num_cores))
    out = k(idx, ent_emb, tblw)
    return out.reshape(B, 1)
